# baseline (speedup 1.0000x reference)
import math

import jax
import jax.numpy as jnp
import numpy as np
from jax.experimental import pallas as pl
from jax.experimental.pallas import tpu as pltpu
from jax.experimental.pallas import tpu_sc as plsc

B, L, D, H = 4, 2048, 1024, 16
DK = D // H
TOPK = int(math.log(L))
F = L // 2 + 1
FPAD = 1152

_t = np.arange(L)
_f = np.arange(FPAD)
_mask = (_f < F)[:, None]
_ang = 2.0 * np.pi * np.outer(_f, _t) / L
CF_MAT = (np.cos(_ang) * _mask).astype(np.float32)
SF_MAT = (np.sin(_ang) * _mask).astype(np.float32)
_af = np.where((_f == 0) | (_f == F - 1), 1.0, 2.0) * (_f < F) / L
_angi = 2.0 * np.pi * np.outer(_t, _f) / L
CI_MAT = (np.cos(_angi) * _af).astype(np.float32)
SI_MAT = (-np.sin(_angi) * _af).astype(np.float32)

RT = 256
FT = 192
TT = 256
VT = 256
NMT = L // VT


def _k1_body(x_ref, w_ref, b_ref, qk_ref, v_ref):
    x = x_ref[...]
    for c in range(2):
        qk_ref[c] = (
            jnp.dot(x, w_ref[c], preferred_element_type=jnp.float32)
            + b_ref[c]
        )
    v = jnp.dot(x, w_ref[2], preferred_element_type=jnp.float32) + b_ref[2]
    for s in range(8):
        v_ref[0, :, s, :] = v[:, s * 128:(s + 1) * 128]


def _k2_body(q_ref, k_ref, cf_ref, sf_ref, pr_ref, pi_ref):
    q = q_ref[0, 0]
    k = k_ref[0, 0]
    cf = cf_ref[...]
    sf = sf_ref[...]
    qr = jnp.dot(cf, q, preferred_element_type=jnp.float32)
    qi = -jnp.dot(sf, q, preferred_element_type=jnp.float32)
    kr = jnp.dot(cf, k, preferred_element_type=jnp.float32)
    ki = -jnp.dot(sf, k, preferred_element_type=jnp.float32)
    pr_ref[0] = qr * kr + qi * ki
    pi_ref[0] = qi * kr - qr * ki


def _k3_body(pr_ref, pi_ref, ci_ref, si_ref, corr_ref, mean_ref):
    corr = jnp.dot(ci_ref[...], pr_ref[0], preferred_element_type=jnp.float32)
    corr += jnp.dot(si_ref[...], pi_ref[0], preferred_element_type=jnp.float32)
    corr_ref[0] = corr
    mean_ref[0, 0, :] = jnp.mean(corr, axis=1)


def _k4a_body(mv_ref, bm_ref):
    bm_ref[...] = jnp.mean(mv_ref[:, 0, :], axis=0, keepdims=True)


def _sc_topk_body(bm_hbm, idx_hbm, bm_s, idx_s, best_s, sem):
    core = jax.lax.axis_index("sc_core")

    @pl.when(core == 0)
    def _():
        pltpu.async_copy(bm_hbm.at[0], bm_s, sem).wait()
        for i in range(TOPK):
            best_s[0] = jnp.float32(-3e38)
            idx_s[i] = jnp.int32(0)

            @pl.loop(0, L)
            def _(l):
                @pl.when(bm_s[l] > best_s[0])
                def _():
                    best_s[0] = bm_s[l]
                    idx_s[i] = l

            bm_s[idx_s[i]] = jnp.float32(-3e38)
        idx_s[TOPK] = jnp.int32(0)
        pltpu.async_copy(idx_s, idx_hbm.at[0], sem).wait()


def _k4b_body(mv_ref, idx_ref, w_ref):
    mv = mv_ref[:, 0, :]
    iota = jax.lax.broadcasted_iota(jnp.int32, (1, L), 1)
    col8 = jax.lax.broadcasted_iota(jnp.int32, (1, 8), 1)
    wacc = jnp.zeros((B, 8), jnp.float32)
    for i in range(TOPK):
        oh = iota == idx_ref[0, i]
        wv = jnp.sum(jnp.where(oh, mv, 0.0), axis=1, keepdims=True)
        wacc = jnp.where(col8 == i, wv, wacc)
    mask = col8 < TOPK
    z = jnp.where(mask, wacc, -jnp.inf)
    z = z - jnp.max(z, axis=1, keepdims=True)
    e = jnp.where(mask, jnp.exp(z), 0.0)
    w_ref[...] = e / jnp.sum(e, axis=1, keepdims=True)


def _k5_body(idx_ref, w_ref, v_ref, wo_ref, bo_ref, o_ref,
             vdbl, delays, flat_ref):
    b = pl.program_id(0)
    p = pl.program_id(1)

    @pl.when(p == 0)
    def _():
        vdbl[0:L] = v_ref[0]
        vdbl[L:2 * L] = v_ref[0]
        for i in range(TOPK):
            w = w_ref[b, i]
            s0 = idx_ref[0, i]
            slab = vdbl[pl.ds(s0, L), :, :]
            if i == 0:
                delays[...] = w * slab
            else:
                delays[...] += w * slab

    @pl.when(p > 0)
    def _():
        mt = p - 1
        for s in range(8):
            flat_ref[:, s * 128:(s + 1) * 128] = \
                delays[pl.ds(mt * VT, VT), s, :]
        o_ref[0] = (
            jnp.dot(flat_ref[...], wo_ref[...],
                    preferred_element_type=jnp.float32)
            + bo_ref[...]
        )


def kernel(x, Wq, bq, Wk, bk, Wv, bv, Wo, bo):
    f32 = jnp.float32
    cf = jnp.asarray(CF_MAT)
    sf = jnp.asarray(SF_MAT)
    ci = jnp.asarray(CI_MAT)
    si = jnp.asarray(SI_MAT)

    w_all = jnp.stack([Wq.T, Wk.T, Wv.T])
    b_all = jnp.stack([bq, bk, bv]).reshape(3, 1, D)
    xf = x.reshape(B * L, D)
    qk, vslab = pl.pallas_call(
        _k1_body,
        grid=(B * L // RT,),
        in_specs=[
            pl.BlockSpec((RT, D), lambda r: (r, 0)),
            pl.BlockSpec((3, D, D), lambda r: (0, 0, 0)),
            pl.BlockSpec((3, 1, D), lambda r: (0, 0, 0)),
        ],
        out_specs=[
            pl.BlockSpec((2, RT, D), lambda r: (0, r, 0)),
            pl.BlockSpec((1, RT, 8, 128), lambda r: (r // 8, r % 8, 0, 0)),
        ],
        out_shape=[
            jax.ShapeDtypeStruct((2, B * L, D), f32),
            jax.ShapeDtypeStruct((B, L, 8, 128), f32),
        ],
    )(xf, w_all, b_all)
    qk4 = qk.reshape(2, B, L, D)

    pr, pi = pl.pallas_call(
        _k2_body,
        grid=(B, FPAD // FT),
        in_specs=[
            pl.BlockSpec((1, 1, L, D), lambda b, ft: (0, b, 0, 0)),
            pl.BlockSpec((1, 1, L, D), lambda b, ft: (1, b, 0, 0)),
            pl.BlockSpec((FT, L), lambda b, ft: (ft, 0)),
            pl.BlockSpec((FT, L), lambda b, ft: (ft, 0)),
        ],
        out_specs=[
            pl.BlockSpec((1, FT, D), lambda b, ft: (b, ft, 0)),
            pl.BlockSpec((1, FT, D), lambda b, ft: (b, ft, 0)),
        ],
        out_shape=[
            jax.ShapeDtypeStruct((B, FPAD, D), f32),
            jax.ShapeDtypeStruct((B, FPAD, D), f32),
        ],
    )(qk4, qk4, cf, sf)

    corr, mean_value = pl.pallas_call(
        _k3_body,
        grid=(B, L // TT),
        in_specs=[
            pl.BlockSpec((1, FPAD, D), lambda b, t: (b, 0, 0)),
            pl.BlockSpec((1, FPAD, D), lambda b, t: (b, 0, 0)),
            pl.BlockSpec((TT, FPAD), lambda b, t: (t, 0)),
            pl.BlockSpec((TT, FPAD), lambda b, t: (t, 0)),
        ],
        out_specs=[
            pl.BlockSpec((1, TT, D), lambda b, t: (b, t, 0)),
            pl.BlockSpec((1, 1, TT), lambda b, t: (b, 0, t)),
        ],
        out_shape=[
            jax.ShapeDtypeStruct((B, L, D), f32),
            jax.ShapeDtypeStruct((B, 1, L), f32),
        ],
    )(pr, pi, ci, si)

    bm = pl.pallas_call(
        _k4a_body,
        in_specs=[pl.BlockSpec((B, 1, L), lambda: (0, 0, 0))],
        out_specs=pl.BlockSpec((1, L), lambda: (0, 0)),
        out_shape=jax.ShapeDtypeStruct((1, L), f32),
    )(mean_value)

    sc_topk = pl.kernel(
        _sc_topk_body,
        out_type=jax.ShapeDtypeStruct((1, 8), jnp.int32),
        mesh=plsc.ScalarSubcoreMesh(axis_name="sc_core", num_cores=2),
        scratch_types=[
            pltpu.SMEM((L,), f32),
            pltpu.SMEM((8,), jnp.int32),
            pltpu.SMEM((1,), f32),
            pltpu.SemaphoreType.DMA,
        ],
    )
    idx = sc_topk(bm)

    w_sm = pl.pallas_call(
        _k4b_body,
        in_specs=[
            pl.BlockSpec((B, 1, L), lambda: (0, 0, 0)),
            pl.BlockSpec(memory_space=pltpu.SMEM),
        ],
        out_specs=pl.BlockSpec((B, 8), lambda: (0, 0)),
        out_shape=jax.ShapeDtypeStruct((B, 8), f32),
    )(mean_value, idx)

    out = pl.pallas_call(
        _k5_body,
        grid=(B, 1 + NMT),
        in_specs=[
            pl.BlockSpec(memory_space=pltpu.SMEM),
            pl.BlockSpec(memory_space=pltpu.SMEM),
            pl.BlockSpec((1, L, 8, 128), lambda b, p: (b, 0, 0, 0)),
            pl.BlockSpec((D, D), lambda b, p: (0, 0)),
            pl.BlockSpec((1, D), lambda b, p: (0, 0)),
        ],
        out_specs=pl.BlockSpec(
            (1, VT, D), lambda b, p: (b, jnp.maximum(p - 1, 0), 0)
        ),
        out_shape=jax.ShapeDtypeStruct((B, L, D), f32),
        scratch_shapes=[
            pltpu.VMEM((2 * L, 8, 128), f32),
            pltpu.VMEM((L, 8, 128), f32),
            pltpu.VMEM((VT, D), f32),
        ],
    )(idx, w_sm, vslab, Wo.T, bo.reshape(1, D))

    attn = corr.reshape(B, L, H, DK)
    return out, attn
